# baseline (speedup 1.0000x reference)
import functools

import jax
import jax.numpy as jnp
from jax import lax
from jax.experimental import pallas as pl
from jax.experimental.pallas import tpu as pltpu
from jax.experimental.pallas import tpu_sc as plsc

N = 10000
E = 320000
D = 128
HID = 256

NC, NS = 2, 16
NW = NC * NS
CHUNK = 128
NBLK = 8
NP = 10240
RPS = NP // NS
NCHUNK = 80
NBLOCKS = NCHUNK // NBLK
EP = NW * NCHUNK * CHUNK

BM = 1024


def _sc_mesh():
    return plsc.VectorSubcoreMesh(core_axis_name="c", subcore_axis_name="s")


@functools.partial(
    pl.kernel,
    out_type=jax.ShapeDtypeStruct((NC * NP, D), jnp.float32),
    mesh=_sc_mesh(),
    scratch_types=[
        pltpu.VMEM((NCHUNK, CHUNK), jnp.int32),
        pltpu.VMEM((CHUNK, D), jnp.float32),
        pltpu.VMEM_SHARED((NP, D), jnp.float32),
    ],
)
def _sc_degree(dst_hbm, out_hbm, dstv, ones_v, cnt_sh):
    cid = lax.axis_index("c")
    sid = lax.axis_index("s")
    wid = sid * NC + cid

    @pl.loop(0, CHUNK)
    def _(i):
        for g in range(D // 16):
            ones_v[i, pl.ds(g * 16, 16)] = jnp.zeros((16,), jnp.float32)

    @pl.loop(0, RPS // CHUNK)
    def _(i):
        pltpu.sync_copy(ones_v, cnt_sh.at[pl.ds(sid * RPS + i * CHUNK, CHUNK)])

    @pl.loop(0, CHUNK)
    def _(i):
        for g in range(D // 16):
            ones_v[i, pl.ds(g * 16, 16)] = jnp.ones((16,), jnp.float32)

    pltpu.sync_copy(dst_hbm.at[wid], dstv)
    plsc.subcore_barrier()

    @pl.loop(0, NCHUNK)
    def _(j):
        pltpu.sync_copy(ones_v, cnt_sh.at[dstv.at[j]], add=True)

    plsc.subcore_barrier()
    pltpu.sync_copy(
        cnt_sh.at[pl.ds(sid * RPS, RPS)],
        out_hbm.at[pl.ds(cid * NP + sid * RPS, RPS)],
    )


CH = 64
NRING = 3
NMACRO = 53
NCH = NRING * NMACRO
EPA = NW * NCH * CH


@functools.partial(
    pl.kernel,
    out_type=jax.ShapeDtypeStruct((NC * NP, D), jnp.float32),
    mesh=_sc_mesh(),
    scratch_types=[
        pltpu.VMEM((NCH, CH), jnp.int32),
        pltpu.VMEM((NRING, CH), jnp.int32),
        pltpu.VMEM((NRING, CH), jnp.int32),
        [pltpu.VMEM((CH, D), jnp.float32) for _ in range(NRING)],
        pltpu.VMEM_SHARED((NP, D), jnp.float32),
        [pltpu.SemaphoreType.DMA for _ in range(NRING)],
        [pltpu.SemaphoreType.DMA for _ in range(NRING)],
    ],
)
def _sc_aggregate(y_hbm, pk_hbm, out_hbm,
                  pkv, sidx, didx, bufs, acc_sh, ssem, gsem):
    cid = lax.axis_index("c")
    sid = lax.axis_index("s")
    wid = sid * NC + cid

    @pl.loop(0, CH)
    def _(i):
        for g in range(D // 16):
            bufs[0][i, pl.ds(g * 16, 16)] = jnp.zeros((16,), jnp.float32)

    @pl.loop(0, RPS // CH)
    def _(i):
        pltpu.sync_copy(bufs[0], acc_sh.at[pl.ds(sid * RPS + i * CH, CH)])

    pltpu.sync_copy(pk_hbm.at[wid], pkv)
    plsc.subcore_barrier()

    def _unpack(c, b):
        for g in range(CH // 16):
            v = pkv[c, pl.ds(g * 16, 16)]
            sidx[b, pl.ds(g * 16, 16)] = lax.shift_right_logical(v, 16)
            didx[b, pl.ds(g * 16, 16)] = lax.bitwise_and(v, 0xFFFF)

    def _gather(c_slot_b, b):
        del c_slot_b
        return pltpu.async_copy(y_hbm.at[sidx.at[b]], bufs[b], gsem[b])

    def _scatter(b):
        return pltpu.async_copy(bufs[b], acc_sh.at[didx.at[b]], ssem[b],
                                add=True)

    def _gwait(b):
        pltpu.make_async_copy(y_hbm.at[sidx.at[b]], bufs[b], gsem[b]).wait()

    def _swait(b):
        pltpu.make_async_copy(bufs[b], acc_sh.at[didx.at[b]], ssem[b]).wait()

    for b in range(NRING):
        _unpack(b, b)
        _gather(b, b)

    @pl.loop(0, NMACRO - 1)
    def _(k):
        c0 = k * NRING
        for b in range(NRING):
            _gwait(b)
            _scatter(b)
        for b in range(NRING):
            _swait(b)
            _unpack(c0 + NRING + b, b)
            _gather(c0 + NRING + b, b)

    for b in range(NRING):
        _gwait(b)
        _scatter(b)
    for b in range(NRING):
        _swait(b)

    plsc.subcore_barrier()
    pltpu.sync_copy(
        acc_sh.at[pl.ds(sid * RPS, RPS)],
        out_hbm.at[pl.ds(cid * NP + sid * RPS, RPS)],
    )


def _tc1_body(cnt_ref, x_ref, dis_ref, y1_ref):
    c = cnt_ref[0] + cnt_ref[1]
    deg = c[:, 0:1] + 1.0
    dis = lax.rsqrt(deg)
    dis_b = jnp.broadcast_to(dis, (BM, D))
    dis_ref[...] = dis_b
    y1_ref[...] = x_ref[...] * dis_b


def _tc2_body(z_ref, y_ref, dis_ref, w1_ref, b1_ref, w2_ref, y2_ref):
    agg = (z_ref[0] + z_ref[1] + y_ref[...]) * dis_ref[...]
    h = jnp.dot(agg, w1_ref[...], preferred_element_type=jnp.float32)
    h = jnp.maximum(h + b1_ref[...], 0.0)
    y2_ref[...] = jnp.dot(h * dis_ref[:, 0:1], w2_ref[...],
                          preferred_element_type=jnp.float32)


def _tc3_body(z_ref, y_ref, dis_ref, b2_ref, out_ref):
    o = (z_ref[0] + z_ref[1] + y_ref[...]) * dis_ref[...] + b2_ref[...]
    m = jnp.max(o, axis=1, keepdims=True)
    s = o - m
    lse = jnp.log(jnp.sum(jnp.exp(s), axis=1, keepdims=True))
    out_ref[...] = s - lse


def _row_spec(width):
    return pl.BlockSpec((BM, width), lambda i: (i, 0))


def _pair_spec(width):
    return pl.BlockSpec((2, BM, width), lambda i: (0, i, 0))


def _full_spec(shape):
    return pl.BlockSpec(shape, lambda i: (0,) * len(shape))


_GRID = NP // BM

_tc1 = pl.pallas_call(
    _tc1_body,
    grid=(_GRID,),
    in_specs=[_pair_spec(D), _row_spec(D)],
    out_specs=[_row_spec(D), _row_spec(D)],
    out_shape=[jax.ShapeDtypeStruct((NP, D), jnp.float32),
               jax.ShapeDtypeStruct((NP, D), jnp.float32)],
)

_tc2 = pl.pallas_call(
    _tc2_body,
    grid=(_GRID,),
    in_specs=[_pair_spec(D), _row_spec(D), _row_spec(D),
              _full_spec((D, HID)), _full_spec((1, HID)), _full_spec((HID, D))],
    out_specs=_row_spec(D),
    out_shape=jax.ShapeDtypeStruct((NP, D), jnp.float32),
)

_tc3 = pl.pallas_call(
    _tc3_body,
    grid=(_GRID,),
    in_specs=[_pair_spec(D), _row_spec(D), _row_spec(D), _full_spec((1, D))],
    out_specs=_row_spec(D),
    out_shape=jax.ShapeDtypeStruct((NP, D), jnp.float32),
)


def kernel(x, edge_index, W1, b1, W2, b2):
    src = edge_index[0].astype(jnp.int32)
    dst = edge_index[1].astype(jnp.int32)
    pad = jnp.full((EP - E,), NP - 1, dtype=jnp.int32)
    pad_a = jnp.full((EPA - E,), NP - 1, dtype=jnp.int32)
    dst3d = jnp.concatenate([dst, pad]).reshape(NW, NCHUNK, CHUNK)
    pk3d = ((jnp.concatenate([src, pad_a]) << 16)
            | jnp.concatenate([dst, pad_a])).reshape(NW, NCH, CH)
    x_pad = jnp.concatenate(
        [x, jnp.zeros((NP - N, D), jnp.float32)], axis=0)

    cnt = _sc_degree(dst3d).reshape(NC, NP, D)
    dis, y1 = _tc1(cnt, x_pad)

    z1 = _sc_aggregate(y1, pk3d).reshape(NC, NP, D)
    y2 = _tc2(z1, y1, dis, W1, b1.reshape(1, HID), W2)

    z2 = _sc_aggregate(y2, pk3d).reshape(NC, NP, D)
    out = _tc3(z2, y2, dis, b2.reshape(1, D))
    return out[:N]


# speedup vs baseline: 1.0652x; 1.0652x over previous
import functools

import jax
import jax.numpy as jnp
from jax import lax
from jax.experimental import pallas as pl
from jax.experimental.pallas import tpu as pltpu
from jax.experimental.pallas import tpu_sc as plsc

N = 10000
E = 320000
D = 128
HID = 256

NC, NS = 2, 16
NW = NC * NS
CHUNK = 128
NBLK = 8
NP = 10240
RPS = NP // NS
NCHUNK = 80
NBLOCKS = NCHUNK // NBLK
EP = NW * NCHUNK * CHUNK

BM = 1024


def _sc_mesh():
    return plsc.VectorSubcoreMesh(core_axis_name="c", subcore_axis_name="s")


@functools.partial(
    pl.kernel,
    out_type=jax.ShapeDtypeStruct((NC * NP, D), jnp.float32),
    mesh=_sc_mesh(),
    scratch_types=[
        pltpu.VMEM((NCHUNK, CHUNK), jnp.int32),
        pltpu.VMEM((CHUNK, D), jnp.float32),
        pltpu.VMEM_SHARED((NP, D), jnp.float32),
    ],
)
def _sc_degree(dst_hbm, out_hbm, dstv, ones_v, cnt_sh):
    cid = lax.axis_index("c")
    sid = lax.axis_index("s")
    wid = sid * NC + cid

    @pl.loop(0, CHUNK)
    def _(i):
        for g in range(D // 16):
            ones_v[i, pl.ds(g * 16, 16)] = jnp.zeros((16,), jnp.float32)

    @pl.loop(0, RPS // CHUNK)
    def _(i):
        pltpu.sync_copy(ones_v, cnt_sh.at[pl.ds(sid * RPS + i * CHUNK, CHUNK)])

    @pl.loop(0, CHUNK)
    def _(i):
        for g in range(D // 16):
            ones_v[i, pl.ds(g * 16, 16)] = jnp.ones((16,), jnp.float32)

    pltpu.sync_copy(dst_hbm.at[wid], dstv)
    plsc.subcore_barrier()

    @pl.loop(0, NCHUNK)
    def _(j):
        pltpu.sync_copy(ones_v, cnt_sh.at[dstv.at[j]], add=True)

    plsc.subcore_barrier()
    pltpu.sync_copy(
        cnt_sh.at[pl.ds(sid * RPS, RPS)],
        out_hbm.at[pl.ds(cid * NP + sid * RPS, RPS)],
    )


CH = 64
NRING = 3
NMACRO = 53
NCH = NRING * NMACRO
EPA = NW * NCH * CH


@functools.partial(
    pl.kernel,
    out_type=jax.ShapeDtypeStruct((NC * NP, D), jnp.float32),
    mesh=_sc_mesh(),
    scratch_types=[
        pltpu.VMEM((NCH, CH), jnp.int32),
        pltpu.VMEM((NRING, CH), jnp.int32),
        pltpu.VMEM((NRING, CH), jnp.int32),
        [pltpu.VMEM((CH, D), jnp.float32) for _ in range(NRING)],
        pltpu.VMEM_SHARED((NP, D), jnp.float32),
        [pltpu.SemaphoreType.DMA for _ in range(NRING)],
        [pltpu.SemaphoreType.DMA for _ in range(NRING)],
    ],
)
def _sc_aggregate(y_hbm, pk_hbm, out_hbm,
                  pkv, sidx, didx, bufs, acc_sh, ssem, gsem):
    cid = lax.axis_index("c")
    sid = lax.axis_index("s")
    wid = sid * NC + (1 - cid)

    @pl.loop(0, CH)
    def _(i):
        for g in range(D // 16):
            bufs[0][i, pl.ds(g * 16, 16)] = jnp.zeros((16,), jnp.float32)

    @pl.loop(0, RPS // CH)
    def _(i):
        pltpu.sync_copy(bufs[0], acc_sh.at[pl.ds(sid * RPS + i * CH, CH)])

    pltpu.sync_copy(pk_hbm.at[wid], pkv)
    plsc.subcore_barrier()

    def _unpack(c, b):
        for g in range(CH // 16):
            v = pkv[c, pl.ds(g * 16, 16)]
            sidx[b, pl.ds(g * 16, 16)] = lax.shift_right_logical(v, 16)
            didx[b, pl.ds(g * 16, 16)] = lax.bitwise_and(v, 0xFFFF)

    def _gather(c_slot_b, b):
        del c_slot_b
        return pltpu.async_copy(y_hbm.at[sidx.at[b]], bufs[b], gsem[b])

    def _scatter(b):
        return pltpu.async_copy(bufs[b], acc_sh.at[didx.at[b]], ssem[b],
                                add=True)

    def _gwait(b):
        pltpu.make_async_copy(y_hbm.at[sidx.at[b]], bufs[b], gsem[b]).wait()

    def _swait(b):
        pltpu.make_async_copy(bufs[b], acc_sh.at[didx.at[b]], ssem[b]).wait()

    for b in range(NRING):
        _unpack(b, b)
        _gather(b, b)

    @pl.loop(0, NMACRO - 1)
    def _(k):
        c0 = k * NRING
        for b in range(NRING):
            _gwait(b)
            _scatter(b)
        for b in range(NRING):
            _swait(b)
            _unpack(c0 + NRING + b, b)
            _gather(c0 + NRING + b, b)

    for b in range(NRING):
        _gwait(b)
        _scatter(b)
    for b in range(NRING):
        _swait(b)

    plsc.subcore_barrier()
    pltpu.sync_copy(
        acc_sh.at[pl.ds(sid * RPS, RPS)],
        out_hbm.at[pl.ds(cid * NP + sid * RPS, RPS)],
    )


def _tc1_body(cnt_ref, x_ref, dis_ref, y1_ref):
    c = cnt_ref[0] + cnt_ref[1]
    deg = c[:, 0:1] + 1.0
    dis = lax.rsqrt(deg)
    dis_b = jnp.broadcast_to(dis, (BM, D))
    dis_ref[...] = dis_b
    y1_ref[...] = x_ref[...] * dis_b


def _tc2_body(z_ref, y_ref, dis_ref, w1_ref, b1_ref, w2_ref, y2_ref):
    agg = (z_ref[0] + z_ref[1] + y_ref[...]) * dis_ref[...]
    h = jnp.dot(agg, w1_ref[...], preferred_element_type=jnp.float32)
    h = jnp.maximum(h + b1_ref[...], 0.0)
    y2_ref[...] = jnp.dot(h * dis_ref[:, 0:1], w2_ref[...],
                          preferred_element_type=jnp.float32)


def _tc3_body(z_ref, y_ref, dis_ref, b2_ref, out_ref):
    o = (z_ref[0] + z_ref[1] + y_ref[...]) * dis_ref[...] + b2_ref[...]
    m = jnp.max(o, axis=1, keepdims=True)
    s = o - m
    lse = jnp.log(jnp.sum(jnp.exp(s), axis=1, keepdims=True))
    out_ref[...] = s - lse


def _row_spec(width):
    return pl.BlockSpec((BM, width), lambda i: (i, 0))


def _pair_spec(width):
    return pl.BlockSpec((2, BM, width), lambda i: (0, i, 0))


def _full_spec(shape):
    return pl.BlockSpec(shape, lambda i: (0,) * len(shape))


_GRID = NP // BM

_tc1 = pl.pallas_call(
    _tc1_body,
    grid=(_GRID,),
    in_specs=[_pair_spec(D), _row_spec(D)],
    out_specs=[_row_spec(D), _row_spec(D)],
    out_shape=[jax.ShapeDtypeStruct((NP, D), jnp.float32),
               jax.ShapeDtypeStruct((NP, D), jnp.float32)],
)

_tc2 = pl.pallas_call(
    _tc2_body,
    grid=(_GRID,),
    in_specs=[_pair_spec(D), _row_spec(D), _row_spec(D),
              _full_spec((D, HID)), _full_spec((1, HID)), _full_spec((HID, D))],
    out_specs=_row_spec(D),
    out_shape=jax.ShapeDtypeStruct((NP, D), jnp.float32),
)

_tc3 = pl.pallas_call(
    _tc3_body,
    grid=(_GRID,),
    in_specs=[_pair_spec(D), _row_spec(D), _row_spec(D), _full_spec((1, D))],
    out_specs=_row_spec(D),
    out_shape=jax.ShapeDtypeStruct((NP, D), jnp.float32),
)


def kernel(x, edge_index, W1, b1, W2, b2):
    src = edge_index[0].astype(jnp.int32)
    dst = edge_index[1].astype(jnp.int32)
    pad = jnp.full((EP - E,), NP - 1, dtype=jnp.int32)
    pad_a = jnp.full((EPA - E,), NP - 1, dtype=jnp.int32)
    dst3d = jnp.concatenate([dst, pad]).reshape(NW, NCHUNK, CHUNK)
    pk3d = ((jnp.concatenate([src, pad_a]) << 16)
            | jnp.concatenate([dst, pad_a])).reshape(NCH, NW, CH).swapaxes(0, 1)
    x_pad = jnp.concatenate(
        [x, jnp.zeros((NP - N, D), jnp.float32)], axis=0)

    cnt = _sc_degree(dst3d).reshape(NC, NP, D)
    dis, y1 = _tc1(cnt, x_pad)

    z1 = _sc_aggregate(y1, pk3d).reshape(NC, NP, D)
    y2 = _tc2(z1, y1, dis, W1, b1.reshape(1, HID), W2)

    z2 = _sc_aggregate(y2, pk3d).reshape(NC, NP, D)
    out = _tc3(z2, y2, dis, b2.reshape(1, D))
    return out[:N]
